# baseline (speedup 1.0000x reference)
import functools

import jax
import jax.numpy as jnp
from jax import lax
from jax.experimental import pallas as pl
from jax.experimental.pallas import tpu as pltpu
from jax.experimental.pallas import tpu_sc as plsc

EMB = 64
BATCH = 16384
SEQ = 50
B = BATCH * SEQ

NC = 2
NS = 16
NW = NC * NS
ROWS_PER_W = B // NW

SUB = 512
NSUB = ROWS_PER_W // SUB
K = 1
CH = SUB * K
NCH = NSUB // K
NB = 2

_mesh = plsc.VectorSubcoreMesh(core_axis_name="c", subcore_axis_name="s")


@functools.partial(
    pl.kernel,
    mesh=_mesh,
    out_type=jax.ShapeDtypeStruct((B, EMB), jnp.float32),
    scratch_types=[
        pltpu.VMEM((NSUB, SUB), jnp.int32),
        pltpu.VMEM((NB, CH, EMB), jnp.float32),
        pltpu.SemaphoreType.DMA,
        pltpu.SemaphoreType.DMA,
        pltpu.SemaphoreType.DMA,
    ],
    compiler_params=pltpu.CompilerParams(use_tc_tiling_on_sc=False),
)
def _gather_kernel(idx_hbm, table_hbm, out_hbm, idx_v, rows_v, gsem, os0, os1):
    wid = lax.axis_index("s") * NC + lax.axis_index("c")
    row_base = wid * NSUB
    osems = (os0, os1)

    pltpu.sync_copy(idx_hbm.at[pl.ds(row_base, NSUB)], idx_v)

    def body(h, carry):
        for b in range(NB):
            c = h * NB + b
            buf = rows_v.at[b]
            out_slc = out_hbm.at[pl.ds((row_base + c * K) * SUB, CH)]

            @pl.when(h > 0)
            def _():
                pltpu.make_async_copy(buf, out_slc, osems[b]).wait()

            copies = [
                pltpu.async_copy(
                    table_hbm.at[idx_v.at[c * K + j]],
                    buf.at[pl.ds(j * SUB, SUB)],
                    gsem,
                )
                for j in range(K)
            ]
            for cp in copies:
                cp.wait()

            pltpu.async_copy(buf, out_slc, osems[b])
        return carry

    lax.fori_loop(0, NCH // NB, body, 0)

    for b in range(NB):
        pltpu.make_async_copy(
            rows_v.at[b], out_hbm.at[pl.ds(row_base * SUB, CH)], osems[b]
        ).wait()


def kernel(data, W_i):
    idx = data.reshape(B // SUB, SUB)
    out = _gather_kernel(idx, W_i)
    return out.reshape(BATCH, SEQ, EMB)
